# Initial kernel scaffold; baseline (speedup 1.0000x reference)
import functools

import jax
import jax.numpy as jnp
from jax import lax
from jax.experimental import pallas as pl
from jax.experimental.pallas import tpu as pltpu
from jax.experimental.pallas import tpu_sc as plsc

NC = 2
NS = 16
NW = NC * NS
CHUNK = 80


def _aggregate_sc(x, src, dst, n_pad):
    n, d = x.shape
    e = src.shape[0]
    epw = e // NW
    n_chunks = epw // CHUNK
    rows_per_tile = n_pad // NS

    mesh = plsc.VectorSubcoreMesh(
        core_axis_name="c", subcore_axis_name="s", num_cores=NC,
        num_subcores=NS)

    @functools.partial(
        pl.kernel,
        out_type=(
            jax.ShapeDtypeStruct((NC, n_pad, d), jnp.float32),
            jax.ShapeDtypeStruct((NC, n_pad), jnp.float32),
        ),
        mesh=mesh,
        scratch_types=[
            pltpu.VMEM((CHUNK,), jnp.int32),
            pltpu.VMEM((CHUNK,), jnp.int32),
            pltpu.VMEM((CHUNK, d), jnp.float32),
            pltpu.VMEM((CHUNK,), jnp.float32),
            pltpu.VMEM_SHARED((n_pad, d), jnp.float32),
            pltpu.VMEM_SHARED((n_pad,), jnp.float32),
            pltpu.SemaphoreType.DMA,
        ],
    )
    def agg(x_hbm, src_hbm, dst_hbm, zrows_hbm, zdeg_hbm,
            nei_out, deg_out,
            src_idx, dst_idx, rows, ones_v, acc, dacc, sem):
        cid = lax.axis_index("c")
        sid = lax.axis_index("s")
        wid = sid * NC + cid

        zsl = pl.ds(sid * rows_per_tile, rows_per_tile)
        pltpu.sync_copy(zrows_hbm, acc.at[zsl])
        pltpu.sync_copy(zdeg_hbm, dacc.at[zsl])
        for j in range(CHUNK // 16):
            ones_v[pl.ds(j * 16, 16)] = jnp.ones((16,), jnp.float32)
        plsc.subcore_barrier()

        def body(i, carry):
            base = pl.multiple_of(wid * epw + i * CHUNK, 8)
            pltpu.sync_copy(src_hbm.at[pl.ds(base, CHUNK)], src_idx)
            pltpu.sync_copy(dst_hbm.at[pl.ds(base, CHUNK)], dst_idx)
            pltpu.async_copy(x_hbm.at[src_idx], rows, sem).wait()
            pltpu.sync_copy(rows, acc.at[dst_idx], add=True)
            pltpu.sync_copy(ones_v, dacc.at[dst_idx], add=True)
            return carry

        lax.fori_loop(0, n_chunks, body, 0)
        plsc.subcore_barrier()

        pltpu.sync_copy(acc.at[zsl], nei_out.at[cid, zsl])
        pltpu.sync_copy(dacc.at[zsl], deg_out.at[cid, zsl])

    zrows = jnp.zeros((rows_per_tile, d), jnp.float32)
    zdeg = jnp.zeros((rows_per_tile,), jnp.float32)
    return agg(x, src, dst, zrows, zdeg)


def _linear_tc(x, nei, deg, w, b, row_block):
    n, d = x.shape

    def body(x_ref, n_ref, d_ref, w_ref, b_ref, o_ref):
        s = x_ref[...] + n_ref[0] + n_ref[1]
        inv = 1.0 / (d_ref[0] + d_ref[1] + 1.0)
        s = s * inv
        o_ref[...] = lax.dot_general(
            s, w_ref[...], (((1,), (1,)), ((), ())),
            preferred_element_type=jnp.float32,
            precision=lax.Precision.HIGHEST) + b_ref[...]

    grid = (n // row_block,)
    return pl.pallas_call(
        body,
        grid=grid,
        in_specs=[
            pl.BlockSpec((row_block, d), lambda i: (i, 0)),
            pl.BlockSpec((NC, row_block, d), lambda i: (0, i, 0)),
            pl.BlockSpec((NC, row_block, 1), lambda i: (0, i, 0)),
            pl.BlockSpec((d, d), lambda i: (0, 0)),
            pl.BlockSpec((1, d), lambda i: (0, 0)),
        ],
        out_specs=pl.BlockSpec((row_block, d), lambda i: (i, 0)),
        out_shape=jax.ShapeDtypeStruct((n, d), jnp.float32),
    )(x, nei, deg, w, b)


def kernel(x, edge_index, W, b):
    n, d = x.shape
    src = edge_index[0]
    dst = edge_index[1]

    n_pad = ((n + 8 * NS - 1) // (8 * NS)) * (8 * NS)
    nei, deg = _aggregate_sc(x, src, dst, n_pad)

    nei = nei[:, :n, :]
    deg = deg[:, :n, None]
    return _linear_tc(x, nei, deg, W, b.reshape(1, d), row_block=1000)


# speedup vs baseline: 5.9879x; 5.9879x over previous
import functools

import jax
import jax.numpy as jnp
from jax import lax
from jax.experimental import pallas as pl
from jax.experimental.pallas import tpu as pltpu
from jax.experimental.pallas import tpu_sc as plsc

NC = 2
NS = 16
NW = NC * NS
CHUNK = 80


def _aggregate_sc(x, src, dst, n_pad):
    n, d = x.shape
    e = src.shape[0]
    epw = e // NW
    n_chunks = epw // CHUNK
    rows_per_tile = n_pad // NS

    mesh = plsc.VectorSubcoreMesh(
        core_axis_name="c", subcore_axis_name="s", num_cores=NC,
        num_subcores=NS)

    @functools.partial(
        pl.kernel,
        out_type=(
            jax.ShapeDtypeStruct((NC, n_pad, d), jnp.float32),
            jax.ShapeDtypeStruct((NC * n_pad,), jnp.float32),
        ),
        mesh=mesh,
        scratch_types=[
            pltpu.VMEM((CHUNK,), jnp.int32),
            pltpu.VMEM((CHUNK,), jnp.int32),
            pltpu.VMEM((CHUNK, d), jnp.float32),
            pltpu.VMEM((CHUNK,), jnp.float32),
            pltpu.VMEM((n_pad,), jnp.float32),
            pltpu.VMEM_SHARED((n_pad, d), jnp.float32),
            pltpu.VMEM_SHARED((n_pad,), jnp.float32),
            pltpu.SemaphoreType.DMA,
        ],
    )
    def agg(x_hbm, src_hbm, dst_hbm, zrows_hbm,
            nei_out, deg_out,
            src_idx, dst_idx, rows, ones_v, dbuf, acc, dacc, sem):
        cid = lax.axis_index("c")
        sid = lax.axis_index("s")
        wid = sid * NC + cid

        zsl = pl.ds(sid * rows_per_tile, rows_per_tile)
        pltpu.sync_copy(zrows_hbm, acc.at[zsl])

        @pl.when(sid == 0)
        def _zero_deg():
            def zb(i, c):
                dbuf[pl.ds(i * 16, 16)] = jnp.zeros((16,), jnp.float32)
                return c
            lax.fori_loop(0, n_pad // 16, zb, 0)
            pltpu.sync_copy(dbuf, dacc)

        for j in range(CHUNK // 16):
            ones_v[pl.ds(j * 16, 16)] = jnp.ones((16,), jnp.float32)
        plsc.subcore_barrier()

        def body(i, carry):
            base = pl.multiple_of(wid * epw + i * CHUNK, 8)
            pltpu.sync_copy(src_hbm.at[pl.ds(base, CHUNK)], src_idx)
            pltpu.sync_copy(dst_hbm.at[pl.ds(base, CHUNK)], dst_idx)
            pltpu.async_copy(x_hbm.at[src_idx], rows, sem).wait()
            pltpu.sync_copy(rows, acc.at[dst_idx], add=True)
            pltpu.sync_copy(ones_v, dacc.at[dst_idx], add=True)
            return carry

        lax.fori_loop(0, n_chunks, body, 0)
        plsc.subcore_barrier()

        pltpu.sync_copy(acc.at[zsl], nei_out.at[cid, zsl])

        @pl.when(sid == 0)
        def _pub_deg():
            pltpu.sync_copy(dacc, dbuf)
            dsl = pl.ds(pl.multiple_of(cid * n_pad, 128), n_pad)
            pltpu.sync_copy(dbuf, deg_out.at[dsl])

    zrows = jnp.zeros((rows_per_tile, d), jnp.float32)
    return agg(x, src, dst, zrows)


def _linear_tc(x, nei, deg, w, b, row_block):
    n, d = x.shape

    def body(x_ref, n_ref, d_ref, w_ref, b_ref, o_ref):
        s = x_ref[...] + n_ref[0] + n_ref[1]
        inv = 1.0 / (d_ref[0] + d_ref[1] + 1.0)
        s = s * inv
        o_ref[...] = lax.dot_general(
            s, w_ref[...], (((1,), (1,)), ((), ())),
            preferred_element_type=jnp.float32,
            precision=lax.Precision.HIGHEST) + b_ref[...]

    grid = (n // row_block,)
    return pl.pallas_call(
        body,
        grid=grid,
        in_specs=[
            pl.BlockSpec((row_block, d), lambda i: (i, 0)),
            pl.BlockSpec((NC, row_block, d), lambda i: (0, i, 0)),
            pl.BlockSpec((NC, row_block, 1), lambda i: (0, i, 0)),
            pl.BlockSpec((d, d), lambda i: (0, 0)),
            pl.BlockSpec((1, d), lambda i: (0, 0)),
        ],
        out_specs=pl.BlockSpec((row_block, d), lambda i: (i, 0)),
        out_shape=jax.ShapeDtypeStruct((n, d), jnp.float32),
    )(x, nei, deg, w, b)


def kernel(x, edge_index, W, b):
    n, d = x.shape
    src = edge_index[0]
    dst = edge_index[1]

    n_pad = ((n + 8 * NS - 1) // (8 * NS)) * (8 * NS)
    nei, deg = _aggregate_sc(x, src, dst, n_pad)

    nei = nei[:, :n, :]
    deg = deg.reshape(NC, n_pad)[:, :n, None]
    return _linear_tc(x, nei, deg, W, b.reshape(1, d), row_block=1000)


# speedup vs baseline: 12.2178x; 2.0404x over previous
import functools

import jax
import jax.numpy as jnp
from jax import lax
from jax.experimental import pallas as pl
from jax.experimental.pallas import tpu as pltpu
from jax.experimental.pallas import tpu_sc as plsc

NC = 2
NS = 16
NW = NC * NS
CHUNK = 80
NBUF = 2


def _aggregate_sc(x, src1, dst3, n_pad):
    n, d = x.shape
    n_chunks = dst3.shape[1]
    epw = n_chunks * CHUNK
    rows_per_tile = n_pad // NS

    mesh = plsc.VectorSubcoreMesh(
        core_axis_name="c", subcore_axis_name="s", num_cores=NC,
        num_subcores=NS)

    @functools.partial(
        pl.kernel,
        out_type=(
            jax.ShapeDtypeStruct((NC, n_pad, d), jnp.float32),
            jax.ShapeDtypeStruct((NC * n_pad,), jnp.float32),
        ),
        mesh=mesh,
        scratch_types=[
            pltpu.VMEM((epw,), jnp.int32),
            pltpu.VMEM((n_chunks, CHUNK), jnp.int32),
            pltpu.VMEM((NBUF, CHUNK, d), jnp.float32),
            pltpu.VMEM((CHUNK,), jnp.float32),
            pltpu.VMEM((640,), jnp.float32),
            pltpu.VMEM_SHARED((n_pad, d), jnp.float32),
            pltpu.VMEM_SHARED((n_pad,), jnp.float32),
            pltpu.SemaphoreType.DMA((NBUF,)),
        ],
    )
    def agg(x_hbm, src_hbm, dst_hbm, zrows_hbm,
            nei_out, deg_out,
            src_all, dst_all, rows, ones_v, dbuf, acc, dacc, gsem):
        cid = lax.axis_index("c")
        sid = lax.axis_index("s")
        wid = sid * NC + cid
        dpt = rows_per_tile

        sbase = pl.multiple_of(wid * epw, 8)
        pltpu.sync_copy(src_hbm.at[pl.ds(sbase, epw)], src_all)
        pltpu.sync_copy(dst_hbm.at[wid], dst_all)

        zsl = pl.ds(sid * rows_per_tile, rows_per_tile)
        pltpu.sync_copy(zrows_hbm, acc.at[zsl])
        for j in range(640 // 16):
            dbuf[pl.ds(j * 16, 16)] = jnp.zeros((16,), jnp.float32)
        pltpu.sync_copy(dbuf.at[pl.ds(0, dpt)], dacc.at[pl.ds(sid * dpt, dpt)])
        for j in range(CHUNK // 16):
            ones_v[pl.ds(j * 16, 16)] = jnp.ones((16,), jnp.float32)
        plsc.subcore_barrier()

        for k in range(NBUF):
            pltpu.async_copy(
                x_hbm.at[src_all.at[pl.ds(k * CHUNK, CHUNK)]],
                rows.at[k], gsem.at[k])

        def body(i, carry):
            b = lax.rem(i, NBUF)
            pltpu.make_async_copy(
                x_hbm.at[src_all.at[pl.ds(0, CHUNK)]],
                rows.at[b], gsem.at[b]).wait()
            pltpu.sync_copy(rows.at[b], acc.at[dst_all.at[i]], add=True)
            pltpu.sync_copy(ones_v, dacc.at[dst_all.at[i]], add=True)
            nxt = i + NBUF

            @pl.when(nxt < n_chunks)
            def _refill():
                nbase = pl.multiple_of(nxt * CHUNK, 8)
                pltpu.async_copy(
                    x_hbm.at[src_all.at[pl.ds(nbase, CHUNK)]],
                    rows.at[b], gsem.at[b])
            return carry

        lax.fori_loop(0, n_chunks, body, 0)
        plsc.subcore_barrier()

        pltpu.sync_copy(acc.at[zsl], nei_out.at[cid, zsl])
        pltpu.sync_copy(dacc.at[pl.ds(sid * dpt, dpt)], dbuf.at[pl.ds(0, dpt)])
        dsl = pl.ds(pl.multiple_of(cid * n_pad + sid * dpt, 8), dpt)
        pltpu.sync_copy(dbuf.at[pl.ds(0, dpt)], deg_out.at[dsl])

    zrows = jnp.zeros((rows_per_tile, d), jnp.float32)
    return agg(x, src1, dst3, zrows)


def _linear_tc(x, nei, deg, w, b, row_block):
    n, d = x.shape

    def body(x_ref, n_ref, d_ref, w_ref, b_ref, o_ref):
        s = x_ref[...] + n_ref[0] + n_ref[1]
        inv = 1.0 / (d_ref[0] + d_ref[1] + 1.0)
        s = s * inv
        o_ref[...] = lax.dot_general(
            s, w_ref[...], (((1,), (1,)), ((), ())),
            preferred_element_type=jnp.float32,
            precision=lax.Precision.HIGHEST) + b_ref[...]

    grid = (n // row_block,)
    return pl.pallas_call(
        body,
        grid=grid,
        in_specs=[
            pl.BlockSpec((row_block, d), lambda i: (i, 0)),
            pl.BlockSpec((NC, row_block, d), lambda i: (0, i, 0)),
            pl.BlockSpec((NC, row_block, 1), lambda i: (0, i, 0)),
            pl.BlockSpec((d, d), lambda i: (0, 0)),
            pl.BlockSpec((1, d), lambda i: (0, 0)),
        ],
        out_specs=pl.BlockSpec((row_block, d), lambda i: (i, 0)),
        out_shape=jax.ShapeDtypeStruct((n, d), jnp.float32),
    )(x, nei, deg, w, b)


def kernel(x, edge_index, W, b):
    n, d = x.shape
    e = edge_index.shape[1]
    src = edge_index[0]
    dst = edge_index[1]

    n_pad = ((n + 8 * NS - 1) // (8 * NS)) * (8 * NS)
    epw = e // NW
    epw_pad = ((epw + CHUNK - 1) // CHUNK) * CHUNK
    padn = epw_pad - epw

    srcw = src.reshape(NW, epw)
    dstw = dst.reshape(NW, epw)
    if padn:
        ps = (jnp.arange(NW, dtype=jnp.int32)[:, None] * 131
              + jnp.arange(padn, dtype=jnp.int32)[None, :] * 7) % n
        pd = jnp.broadcast_to(
            n + jnp.arange(NW, dtype=jnp.int32)[:, None], (NW, padn))
        srcw = jnp.concatenate([srcw, ps], axis=1)
        dstw = jnp.concatenate([dstw, pd.astype(jnp.int32)], axis=1)
    src1 = srcw.reshape(NW * epw_pad)
    dst3 = dstw.reshape(NW, epw_pad // CHUNK, CHUNK)

    nei, deg = _aggregate_sc(x, src1, dst3, n_pad)

    nei = nei[:, :n, :]
    deg = deg.reshape(NC, n_pad)[:, :n, None]
    return _linear_tc(x, nei, deg, W, b.reshape(1, d), row_block=1000)
